# baseline (speedup 1.0000x reference)
import functools
import jax
import jax.numpy as jnp
from jax import lax
from jax.experimental import pallas as pl
from jax.experimental.pallas import tpu as pltpu
from jax.experimental.pallas import tpu_sc as plsc

N_CLS_CTX = 4
CTX_DIM = 512
SEQ_LEN = 77
N_PRE = N_CLS_CTX + 1
N_SUF = SEQ_LEN - 2 * N_CLS_CTX - 1


def kernel(vehicle_ids, tokenized_prompts, token_embedding, cls_ctx):
    B = tokenized_prompts.shape[0]
    info = plsc.get_sparse_core_info()
    nc, ns = info.num_cores, info.num_subcores
    nw = nc * ns
    n_per_w = B // nw

    tp = tokenized_prompts.astype(jnp.int32)
    tp_pre = tp[:, :N_PRE]
    tp_suf = tp[:, N_PRE + N_CLS_CTX:]
    cls2d = cls_ctx.reshape(cls_ctx.shape[0] * N_CLS_CTX, CTX_DIM)
    vid4 = (vehicle_ids.astype(jnp.int32)[:, None] * N_CLS_CTX
            + jnp.arange(N_CLS_CTX, dtype=jnp.int32)[None, :])

    mesh = plsc.VectorSubcoreMesh(core_axis_name="c", subcore_axis_name="s")

    @functools.partial(
        pl.kernel,
        mesh=mesh,
        compiler_params=pltpu.CompilerParams(use_tc_tiling_on_sc=False),
        out_type=jax.ShapeDtypeStruct((B, SEQ_LEN, CTX_DIM), jnp.float32),
        scratch_types=[
            pltpu.VMEM((n_per_w, N_PRE), jnp.int32),
            pltpu.VMEM((n_per_w, N_SUF), jnp.int32),
            pltpu.VMEM((n_per_w, N_CLS_CTX), jnp.int32),
            pltpu.VMEM((SEQ_LEN, CTX_DIM), jnp.float32),
            pltpu.VMEM((SEQ_LEN, CTX_DIM), jnp.float32),
            pltpu.SemaphoreType.DMA,
            pltpu.SemaphoreType.DMA,
            pltpu.SemaphoreType.DMA,
        ],
    )
    def prompt_gather(pre_hbm, suf_hbm, vid4_hbm, te_hbm, cls_hbm, out_hbm,
                      pre_v, suf_v, vid4_v, rows0, rows1, gsem, ssem0, ssem1):
        wid = lax.axis_index("s") * nc + lax.axis_index("c")
        base = wid * n_per_w
        pltpu.sync_copy(pre_hbm.at[pl.ds(base, n_per_w), :], pre_v)
        pltpu.sync_copy(suf_hbm.at[pl.ds(base, n_per_w), :], suf_v)
        pltpu.sync_copy(vid4_hbm.at[pl.ds(base, n_per_w), :], vid4_v)

        def one_item(k, i, rows_v, ssem):
            b = base + i
            @pl.when(k > 0)
            def _():
                pltpu.make_async_copy(rows_v, out_hbm.at[b], ssem).wait()
            g1 = pltpu.async_copy(te_hbm.at[pre_v.at[i]],
                                  rows_v.at[pl.ds(0, N_PRE)], gsem)
            g2 = pltpu.async_copy(cls_hbm.at[vid4_v.at[i]],
                                  rows_v.at[pl.ds(N_PRE, N_CLS_CTX)], gsem)
            g3 = pltpu.async_copy(te_hbm.at[suf_v.at[i]],
                                  rows_v.at[pl.ds(N_PRE + N_CLS_CTX, N_SUF)],
                                  gsem)
            g1.wait()
            g2.wait()
            g3.wait()
            pltpu.async_copy(rows_v, out_hbm.at[b], ssem)

        def body(k, carry):
            one_item(k, 2 * k, rows0, ssem0)
            one_item(k, 2 * k + 1, rows1, ssem1)
            return carry

        lax.fori_loop(0, n_per_w // 2, body, 0)
        pltpu.make_async_copy(rows0, out_hbm.at[base], ssem0).wait()
        pltpu.make_async_copy(rows1, out_hbm.at[base], ssem1).wait()

    return prompt_gather(tp_pre, tp_suf, vid4, token_embedding, cls2d)
